# baseline (speedup 1.0000x reference)
import functools

import jax
import jax.numpy as jnp
from jax import lax
from jax.experimental import pallas as pl
from jax.experimental.pallas import tpu as pltpu
from jax.experimental.pallas import tpu_sc as plsc

_VOCAB = 100000
_EMBED_DIM = 128
_BATCH = 16384

_NC = 2
_NS = 16
_CHUNK = 128
_BLOCK = 1024
_C0 = 576
_C1 = _BLOCK - _C0

_mesh = plsc.VectorSubcoreMesh(core_axis_name="c", subcore_axis_name="s")


@functools.partial(
    pl.kernel,
    mesh=_mesh,
    out_type=jax.ShapeDtypeStruct((_BATCH, _EMBED_DIM), jnp.float32),
    scratch_types=[
        pltpu.VMEM((_BLOCK // _CHUNK, _CHUNK), jnp.int32),
        pltpu.VMEM((_C0, _EMBED_DIM), jnp.float32),
        pltpu.SemaphoreType.DMA,
    ],
)
def _gather_kernel(token_hbm, table_hbm, out_hbm, idx_v, rows_v, sem):
    s = lax.axis_index("s")
    c = lax.axis_index("c")
    n_rows = _BLOCK // _CHUNK
    pltpu.sync_copy(token_hbm.at[pl.ds(s * n_rows, n_rows)], idx_v)

    @pl.when(c == 0)
    def _():
        copies = []
        for j in range(4):
            copies.append(pltpu.async_copy(
                table_hbm.at[idx_v.at[j]],
                rows_v.at[pl.ds(j * _CHUNK, _CHUNK)], sem))
        copies.append(pltpu.async_copy(
            table_hbm.at[idx_v.at[4, pl.ds(0, 64)]],
            rows_v.at[pl.ds(4 * _CHUNK, 64)], sem))
        for cp in copies:
            cp.wait()
        pltpu.sync_copy(rows_v.at[pl.ds(0, _C0)],
                        out_hbm.at[pl.ds(s * _BLOCK, _C0)])

    @pl.when(c == 1)
    def _():
        copies = [pltpu.async_copy(
            table_hbm.at[idx_v.at[4, pl.ds(64, 64)]],
            rows_v.at[pl.ds(0, 64)], sem)]
        for j in range(3):
            copies.append(pltpu.async_copy(
                table_hbm.at[idx_v.at[5 + j]],
                rows_v.at[pl.ds(64 + j * _CHUNK, _CHUNK)], sem))
        for cp in copies:
            cp.wait()
        pltpu.sync_copy(rows_v.at[pl.ds(0, _C1)],
                        out_hbm.at[pl.ds(s * _BLOCK + _C0, _C1)])


def kernel(token, table):
    idx2d = token.reshape(_BATCH // _CHUNK, _CHUNK)
    out = _gather_kernel(idx2d, table)
    return out.reshape(_BATCH, 1, _EMBED_DIM)


# speedup vs baseline: 1.0403x; 1.0403x over previous
import functools

import jax
import jax.numpy as jnp
from jax import lax
from jax.experimental import pallas as pl
from jax.experimental.pallas import tpu as pltpu
from jax.experimental.pallas import tpu_sc as plsc

_VOCAB = 100000
_EMBED_DIM = 128
_BATCH = 16384

_NC = 2
_NS = 16
_CHUNK = 128
_BLOCK = 1024
_C0 = 448
_C1 = _BLOCK - _C0

_mesh = plsc.VectorSubcoreMesh(core_axis_name="c", subcore_axis_name="s")


@functools.partial(
    pl.kernel,
    mesh=_mesh,
    out_type=jax.ShapeDtypeStruct((_BATCH, _EMBED_DIM), jnp.float32),
    scratch_types=[
        pltpu.VMEM((_BLOCK // _CHUNK, _CHUNK), jnp.int32),
        pltpu.VMEM((_C1, _EMBED_DIM), jnp.float32),
        pltpu.SemaphoreType.DMA,
    ],
)
def _gather_kernel(token_hbm, table_hbm, out_hbm, idx_v, rows_v, sem):
    s = lax.axis_index("s")
    c = lax.axis_index("c")
    n_rows = _BLOCK // _CHUNK
    pltpu.sync_copy(token_hbm.at[pl.ds(s * n_rows, n_rows)], idx_v)

    @pl.when(c == 0)
    def _():
        copies = []
        for j in range(3):
            copies.append(pltpu.async_copy(
                table_hbm.at[idx_v.at[j]],
                rows_v.at[pl.ds(j * _CHUNK, _CHUNK)], sem))
        copies.append(pltpu.async_copy(
            table_hbm.at[idx_v.at[3, pl.ds(0, 64)]],
            rows_v.at[pl.ds(3 * _CHUNK, 64)], sem))
        for cp in copies:
            cp.wait()
        pltpu.sync_copy(rows_v.at[pl.ds(0, _C0)],
                        out_hbm.at[pl.ds(s * _BLOCK, _C0)])

    @pl.when(c == 1)
    def _():
        copies = [pltpu.async_copy(
            table_hbm.at[idx_v.at[3, pl.ds(64, 64)]],
            rows_v.at[pl.ds(0, 64)], sem)]
        for j in range(4):
            copies.append(pltpu.async_copy(
                table_hbm.at[idx_v.at[4 + j]],
                rows_v.at[pl.ds(64 + j * _CHUNK, _CHUNK)], sem))
        for cp in copies:
            cp.wait()
        pltpu.sync_copy(rows_v.at[pl.ds(0, _C1)],
                        out_hbm.at[pl.ds(s * _BLOCK + _C0, _C1)])


def kernel(token, table):
    idx2d = token.reshape(_BATCH // _CHUNK, _CHUNK)
    out = _gather_kernel(idx2d, table)
    return out.reshape(_BATCH, 1, _EMBED_DIM)


# speedup vs baseline: 1.0455x; 1.0050x over previous
import functools

import jax
import jax.numpy as jnp
from jax import lax
from jax.experimental import pallas as pl
from jax.experimental.pallas import tpu as pltpu
from jax.experimental.pallas import tpu_sc as plsc

_VOCAB = 100000
_EMBED_DIM = 128
_BATCH = 16384

_NC = 2
_NS = 16
_NW = _NC * _NS
_B_PER_W = _BATCH // _NW
_CHUNK = 128
_N_CHUNKS = _B_PER_W // _CHUNK

_mesh = plsc.VectorSubcoreMesh(core_axis_name="c", subcore_axis_name="s")


@functools.partial(
    pl.kernel,
    mesh=_mesh,
    out_type=jax.ShapeDtypeStruct((_BATCH, _EMBED_DIM), jnp.float32),
    scratch_types=[
        pltpu.VMEM((_N_CHUNKS, _CHUNK), jnp.int32),
        pltpu.VMEM((_B_PER_W, _EMBED_DIM), jnp.float32),
        pltpu.SemaphoreType.DMA,
    ],
)
def _gather_kernel(token_hbm, table_hbm, out_hbm, idx_v, rows_v, sem):
    wid = lax.axis_index("s") * _NC + lax.axis_index("c")
    pltpu.sync_copy(token_hbm.at[pl.ds(wid * _N_CHUNKS, _N_CHUNKS)], idx_v)
    copies = []
    for j in range(_N_CHUNKS):
        copies.append(
            pltpu.async_copy(
                table_hbm.at[idx_v.at[j]],
                rows_v.at[pl.ds(j * _CHUNK, _CHUNK)],
                sem,
            )
        )
    for c in copies:
        c.wait()
    pltpu.sync_copy(rows_v, out_hbm.at[pl.ds(wid * _B_PER_W, _B_PER_W)])


def kernel(token, table):
    idx2d = token.reshape(_BATCH // _CHUNK, _CHUNK)
    out = _gather_kernel(idx2d, table)
    return out.reshape(_BATCH, 1, _EMBED_DIM)
